# baseline (speedup 1.0000x reference)
import functools

import jax
import jax.numpy as jnp
from jax import lax
from jax.experimental import pallas as pl
from jax.experimental.pallas import tpu as pltpu
from jax.experimental.pallas import tpu_sc as plsc

T = 2048
D = 1024
H = 2048
NE = 8
CAP = 768
SLOTS = NE * CAP
HSLOTS = SLOTS // 2
DC = 128
NCH = D // DC
BUFROWS = 3200
ORW = HSLOTS // 16
TPT = T // 16
TPW = T // 32
BH = 1024


def _route_body(x_ref, g_ref, gw_ref, xn_ref, daA1_ref, daA2_ref,
                daB1_ref, daB2_ref, ga1_ref, ga2_ref, cw1_ref, cw2_ref):
    x = x_ref[...]
    ssq = jnp.sum(x * x, axis=1, keepdims=True)
    xn = x * lax.rsqrt(ssq / D + 1e-6) * g_ref[...]
    xn_ref[...] = xn
    logits = lax.dot_general(gw_ref[...], xn, (((1,), (1,)), ((), ())),
                             preferred_element_type=jnp.float32)
    m = jnp.max(logits, axis=0, keepdims=True)
    ex = jnp.exp(logits - m)
    probs = ex / jnp.sum(ex, axis=0, keepdims=True)
    iota = lax.broadcasted_iota(jnp.int32, (NE, T), 0)
    p1 = jnp.max(probs, axis=0, keepdims=True)
    e1 = jnp.min(jnp.where(probs == p1, iota, NE), axis=0, keepdims=True)
    probs2 = jnp.where(iota == e1, -1.0, probs)
    p2 = jnp.max(probs2, axis=0, keepdims=True)
    e2 = jnp.min(jnp.where(probs2 == p2, iota, NE), axis=0, keepdims=True)
    denom = p1 + p2 + 1e-10
    w1v = p1 / denom
    w2v = p2 / denom
    oh1 = (iota == e1).astype(jnp.float32)
    oh2 = (iota == e2).astype(jnp.float32)

    NB, BSZ = 16, T // 16
    bi = lax.broadcasted_iota(jnp.int32, (BSZ, BSZ), 0)
    bj = lax.broadcasted_iota(jnp.int32, (BSZ, BSZ), 1)
    triu = (bi <= bj).astype(jnp.float32)

    def cumsum0(a):
        ab = a.reshape(NE, NB, BSZ)
        cin = lax.dot_general(ab, triu, (((2,), (0,)), ((), ())))
        tot = cin[:, :, BSZ - 1]
        s = 1
        while s < NB:
            tot_sh = jnp.concatenate(
                [jnp.zeros((NE, s), jnp.float32), tot[:, :NB - s]], axis=1)
            tot = tot + tot_sh
            s *= 2
        pref = jnp.concatenate(
            [jnp.zeros((NE, 1), jnp.float32), tot[:, :NB - 1]], axis=1)
        return (cin + pref[:, :, None]).reshape(NE, T)

    pos1 = jnp.sum(cumsum0(oh1) * oh1, axis=0, keepdims=True).astype(jnp.int32) - 1
    pos2 = jnp.sum(cumsum0(oh2) * oh2, axis=0, keepdims=True).astype(jnp.int32) - 1
    ok1 = pos1 < CAP
    ok2 = pos2 < CAP
    slot1 = e1 * CAP + pos1
    slot2 = e2 * CAP + pos2
    daA1_ref[...] = jnp.where(ok1 & (slot1 < HSLOTS), slot1, HSLOTS)
    daA2_ref[...] = jnp.where(ok2 & (slot2 < HSLOTS), slot2, HSLOTS)
    daB1_ref[...] = jnp.where(ok1 & (slot1 >= HSLOTS), slot1 - HSLOTS, HSLOTS)
    daB2_ref[...] = jnp.where(ok2 & (slot2 >= HSLOTS), slot2 - HSLOTS, HSLOTS)
    ga1_ref[...] = jnp.where(ok1, slot1, 0)
    ga2_ref[...] = jnp.where(ok2, slot2, 0)
    cw1_ref[...] = jnp.where(ok1, w1v, 0.0)
    cw2_ref[...] = jnp.where(ok2, w2v, 0.0)


_route = pl.pallas_call(
    _route_body,
    out_shape=[
        jax.ShapeDtypeStruct((T, D), jnp.float32),
        jax.ShapeDtypeStruct((1, T), jnp.int32),
        jax.ShapeDtypeStruct((1, T), jnp.int32),
        jax.ShapeDtypeStruct((1, T), jnp.int32),
        jax.ShapeDtypeStruct((1, T), jnp.int32),
        jax.ShapeDtypeStruct((1, T), jnp.int32),
        jax.ShapeDtypeStruct((1, T), jnp.int32),
        jax.ShapeDtypeStruct((1, T), jnp.float32),
        jax.ShapeDtypeStruct((1, T), jnp.float32),
    ],
)


@functools.cache
def _make_dispatch():
    mesh = plsc.VectorSubcoreMesh(core_axis_name="c", subcore_axis_name="s")

    @functools.partial(
        pl.kernel,
        mesh=mesh,
        out_type=jax.ShapeDtypeStruct((HSLOTS, D), jnp.float32),
        scratch_types=[
            pltpu.VMEM((TPT, DC), jnp.float32),
            pltpu.VMEM((TPT,), jnp.int32),
            pltpu.VMEM((TPT,), jnp.int32),
            pltpu.VMEM_SHARED((BUFROWS, DC), jnp.float32),
            pltpu.SemaphoreType.DMA,
            pltpu.SemaphoreType.DMA,
            pltpu.SemaphoreType.DMA,
        ],
    )
    def _dispatch(xn_hbm, da1_hbm, da2_hbm, zeros_hbm, ei_hbm,
                  rows_v, idx1_v, idx2_v, shared, sem0, sem1, semr):
        c = lax.axis_index("c")
        s = lax.axis_index("s")
        base_t = s * TPT
        wbase = s * ORW
        pltpu.sync_copy(da1_hbm.at[pl.ds(base_t, TPT)], idx1_v)
        pltpu.sync_copy(da2_hbm.at[pl.ds(base_t, TPT)], idx2_v)
        pending = None
        for cj in range(NCH // 2):
            j = c * (NCH // 2) + cj
            if pending is not None:
                pending.wait()
            cp_z = pltpu.async_copy(zeros_hbm.at[pl.ds(wbase, ORW)],
                                    shared.at[pl.ds(wbase, ORW)], sem0)
            cp_l = pltpu.async_copy(
                xn_hbm.at[pl.ds(base_t, TPT), pl.ds(j * DC, DC)],
                rows_v, sem1)
            cp_z.wait()
            cp_l.wait()
            plsc.subcore_barrier()
            c1 = pltpu.async_copy(rows_v, shared.at[idx1_v], sem0, add=True)
            c2 = pltpu.async_copy(rows_v, shared.at[idx2_v], sem1, add=True)
            c1.wait()
            c2.wait()
            plsc.subcore_barrier()
            pending = pltpu.async_copy(
                shared.at[pl.ds(wbase, ORW)],
                ei_hbm.at[pl.ds(wbase, ORW), pl.ds(j * DC, DC)], semr)
        pending.wait()

    return _dispatch


def _rnd16(v):
    u = lax.bitcast_convert_type(v, jnp.uint32)
    r = u + jnp.uint32(0x7FFF) + ((u >> 16) & jnp.uint32(1))
    return r >> 16


def _pack16(lo, hi):
    w = _rnd16(lo) | (_rnd16(hi) << 16)
    return lax.bitcast_convert_type(w, jnp.int32)


def _unpack16(p):
    w = lax.bitcast_convert_type(p, jnp.uint32)
    lo = lax.bitcast_convert_type(w << 16, jnp.float32)
    hi = lax.bitcast_convert_type(w & jnp.uint32(0xFFFF0000), jnp.float32)
    return lo, hi

def _ffn_body(ei_ref, w1_ref, w2_ref, w3_ref, out_ref, acc_ref):
    hb = pl.program_id(1)
    x = ei_ref[...]
    h1 = lax.dot_general(x, w1_ref[0], (((1,), (0,)), ((), ())),
                         preferred_element_type=jnp.float32)
    h2 = lax.dot_general(x, w2_ref[0], (((1,), (0,)), ((), ())),
                         preferred_element_type=jnp.float32)
    h = h1 * (1.0 / (1.0 + jnp.exp(-h1))) * h2
    part = lax.dot_general(h, w3_ref[0], (((1,), (0,)), ((), ())),
                           preferred_element_type=jnp.float32)

    @pl.when(hb == 0)
    def _():
        acc_ref[...] = part

    @pl.when(hb != 0)
    def _():
        tot = acc_ref[...] + part
        out_ref[...] = _pack16(tot[:, :D // 2], tot[:, D // 2:])


def _ffn_body_aliased(ei_ref, w1_ref, w2_ref, w3_ref, prev_ref, out_ref,
                      acc_ref):
    del prev_ref
    _ffn_body(ei_ref, w1_ref, w2_ref, w3_ref, out_ref, acc_ref)


def _make_ffn(eoff, aliased):
    in_specs = [
        pl.BlockSpec((CAP, D), lambda e, hb: (e, 0)),
        pl.BlockSpec((1, D, BH), lambda e, hb, _o=eoff: (e + _o, 0, hb)),
        pl.BlockSpec((1, D, BH), lambda e, hb, _o=eoff: (e + _o, 0, hb)),
        pl.BlockSpec((1, BH, D), lambda e, hb, _o=eoff: (e + _o, hb, 0)),
    ]
    if aliased:
        in_specs.append(pl.BlockSpec(memory_space=pltpu.MemorySpace.HBM))
    return pl.pallas_call(
        _ffn_body_aliased if aliased else _ffn_body,
        grid=(NE // 2, H // BH),
        in_specs=in_specs,
        out_specs=pl.BlockSpec((CAP, D // 2),
                               lambda e, hb, _o=eoff: (e + _o, 0)),
        out_shape=jax.ShapeDtypeStruct((SLOTS, D // 2), jnp.int32),
        scratch_shapes=[pltpu.VMEM((CAP, D), jnp.float32)],
        input_output_aliases={4: 0} if aliased else {},
    )


_ffn_a = _make_ffn(0, False)
_ffn_b = _make_ffn(NE // 2, True)


@functools.cache
def _make_combine_gather():
    mesh = plsc.VectorSubcoreMesh(core_axis_name="c", subcore_axis_name="s")

    @functools.partial(
        pl.kernel,
        mesh=mesh,
        out_type=jax.ShapeDtypeStruct((2 * T, D // 2), jnp.int32),
        scratch_types=[
            pltpu.VMEM((TPW,), jnp.int32),
            pltpu.VMEM((TPW, D // 2), jnp.int32),
            pltpu.SemaphoreType.DMA,
        ],
    )
    def _combine_gather(eo_hbm, ga1_hbm, ga2_hbm, g_hbm, idx_v, rows_v, sem):
        c = lax.axis_index("c")
        s = lax.axis_index("s")
        wid = s * 2 + c
        base = wid * TPW
        pltpu.sync_copy(ga1_hbm.at[pl.ds(base, TPW)], idx_v)
        pltpu.async_copy(eo_hbm.at[idx_v], rows_v, sem).wait()
        pltpu.sync_copy(rows_v, g_hbm.at[pl.ds(base, TPW)])
        pltpu.sync_copy(ga2_hbm.at[pl.ds(base, TPW)], idx_v)
        pltpu.async_copy(eo_hbm.at[idx_v], rows_v, sem).wait()
        pltpu.sync_copy(rows_v, g_hbm.at[pl.ds(T + base, TPW)])

    return _combine_gather


def _wadd_body(g_ref, cw1_ref, cw2_ref, out_ref):
    lo1, hi1 = _unpack16(g_ref[0])
    lo2, hi2 = _unpack16(g_ref[1])
    out_ref[:, :D // 2] = cw1_ref[...] * lo1 + cw2_ref[...] * lo2
    out_ref[:, D // 2:] = cw1_ref[...] * hi1 + cw2_ref[...] * hi2


_wadd = pl.pallas_call(
    _wadd_body,
    out_shape=jax.ShapeDtypeStruct((T, D), jnp.float32),
)


def kernel(x, norm_g, gate_w, w1, w2, w3):
    b, s, d = x.shape
    xf = x.reshape(T, D)
    (xn, daA1, daA2, daB1, daB2,
     ga1, ga2, cw1, cw2) = _route(xf, norm_g.reshape(1, D), gate_w)
    zeros = jnp.zeros((HSLOTS, DC), jnp.float32)
    disp = _make_dispatch()
    eiA = disp(xn, daA1.reshape(T), daA2.reshape(T), zeros)
    eiB = disp(xn, daB1.reshape(T), daB2.reshape(T), zeros)
    eoA = _ffn_a(eiA, w1, w2, w3)
    eo = _ffn_b(eiB, w1, w2, w3, eoA)
    g = _make_combine_gather()(eo, ga1.reshape(T), ga2.reshape(T))
    out = _wadd(g.reshape(2, T, D // 2), cw1.reshape(T, 1), cw2.reshape(T, 1))
    return out.reshape(b, s, d)


# speedup vs baseline: 1.0161x; 1.0161x over previous
import functools

import jax
import jax.numpy as jnp
from jax import lax
from jax.experimental import pallas as pl
from jax.experimental.pallas import tpu as pltpu
from jax.experimental.pallas import tpu_sc as plsc

T = 2048
D = 1024
H = 2048
NE = 8
CAP = 768
SLOTS = NE * CAP
DC = 128
NCH = D // DC
BUFROWS = 6400
ORW = SLOTS // 16
TPT = T // 16
TPW = T // 32
BH = 1024


def _route_body(x_ref, g_ref, gw_ref, xn_ref, da1_ref, da2_ref,
                ga1_ref, ga2_ref, cw1_ref, cw2_ref):
    x = x_ref[...]
    ssq = jnp.sum(x * x, axis=1, keepdims=True)
    xn = x * lax.rsqrt(ssq / D + 1e-6) * g_ref[...]
    xn_ref[...] = xn
    logits = lax.dot_general(gw_ref[...], xn, (((1,), (1,)), ((), ())),
                             preferred_element_type=jnp.float32)
    m = jnp.max(logits, axis=0, keepdims=True)
    ex = jnp.exp(logits - m)
    probs = ex / jnp.sum(ex, axis=0, keepdims=True)
    iota = lax.broadcasted_iota(jnp.int32, (NE, T), 0)
    p1 = jnp.max(probs, axis=0, keepdims=True)
    e1 = jnp.min(jnp.where(probs == p1, iota, NE), axis=0, keepdims=True)
    probs2 = jnp.where(iota == e1, -1.0, probs)
    p2 = jnp.max(probs2, axis=0, keepdims=True)
    e2 = jnp.min(jnp.where(probs2 == p2, iota, NE), axis=0, keepdims=True)
    denom = p1 + p2 + 1e-10
    w1v = p1 / denom
    w2v = p2 / denom
    oh1 = (iota == e1).astype(jnp.float32)
    oh2 = (iota == e2).astype(jnp.float32)

    NB, BSZ = 16, T // 16
    bi = lax.broadcasted_iota(jnp.int32, (BSZ, BSZ), 0)
    bj = lax.broadcasted_iota(jnp.int32, (BSZ, BSZ), 1)
    triu = (bi <= bj).astype(jnp.float32)

    def cumsum0(a):
        ab = a.reshape(NE, NB, BSZ)
        cin = lax.dot_general(ab, triu, (((2,), (0,)), ((), ())))
        tot = cin[:, :, BSZ - 1]
        s = 1
        while s < NB:
            tot_sh = jnp.concatenate(
                [jnp.zeros((NE, s), jnp.float32), tot[:, :NB - s]], axis=1)
            tot = tot + tot_sh
            s *= 2
        pref = jnp.concatenate(
            [jnp.zeros((NE, 1), jnp.float32), tot[:, :NB - 1]], axis=1)
        return (cin + pref[:, :, None]).reshape(NE, T)

    pos1 = jnp.sum(cumsum0(oh1) * oh1, axis=0, keepdims=True).astype(jnp.int32) - 1
    pos2 = jnp.sum(cumsum0(oh2) * oh2, axis=0, keepdims=True).astype(jnp.int32) - 1
    ok1 = pos1 < CAP
    ok2 = pos2 < CAP
    da1_ref[...] = jnp.where(ok1, e1 * CAP + pos1, SLOTS)
    da2_ref[...] = jnp.where(ok2, e2 * CAP + pos2, SLOTS)
    ga1_ref[...] = jnp.where(ok1, e1 * CAP + pos1, 0)
    ga2_ref[...] = jnp.where(ok2, e2 * CAP + pos2, 0)
    cw1_ref[...] = jnp.where(ok1, w1v, 0.0)
    cw2_ref[...] = jnp.where(ok2, w2v, 0.0)


_route = pl.pallas_call(
    _route_body,
    out_shape=[
        jax.ShapeDtypeStruct((T, D), jnp.float32),
        jax.ShapeDtypeStruct((1, T), jnp.int32),
        jax.ShapeDtypeStruct((1, T), jnp.int32),
        jax.ShapeDtypeStruct((1, T), jnp.int32),
        jax.ShapeDtypeStruct((1, T), jnp.int32),
        jax.ShapeDtypeStruct((1, T), jnp.float32),
        jax.ShapeDtypeStruct((1, T), jnp.float32),
    ],
)


@functools.cache
def _make_dispatch():
    mesh = plsc.VectorSubcoreMesh(core_axis_name="c", subcore_axis_name="s")

    @functools.partial(
        pl.kernel,
        mesh=mesh,
        out_type=jax.ShapeDtypeStruct((SLOTS, D), jnp.float32),
        scratch_types=[
            pltpu.VMEM((TPT, DC), jnp.float32),
            pltpu.VMEM((TPT,), jnp.int32),
            pltpu.VMEM((TPT,), jnp.int32),
            pltpu.VMEM_SHARED((BUFROWS, DC), jnp.float32),
            pltpu.SemaphoreType.DMA,
            pltpu.SemaphoreType.DMA,
            pltpu.SemaphoreType.DMA,
        ],
    )
    def _dispatch(xn_hbm, da1_hbm, da2_hbm, zeros_hbm, ei_hbm,
                  rows_v, idx1_v, idx2_v, shared, sem0, sem1, semr):
        c = lax.axis_index("c")
        s = lax.axis_index("s")
        base_t = s * TPT
        wbase = s * ORW
        pltpu.sync_copy(da1_hbm.at[pl.ds(base_t, TPT)], idx1_v)
        pltpu.sync_copy(da2_hbm.at[pl.ds(base_t, TPT)], idx2_v)
        pending = None
        for cj in range(NCH // 2):
            j = c * (NCH // 2) + cj
            if pending is not None:
                pending.wait()
            cp_z = pltpu.async_copy(zeros_hbm.at[pl.ds(wbase, ORW)],
                                    shared.at[pl.ds(wbase, ORW)], sem0)
            cp_l = pltpu.async_copy(
                xn_hbm.at[pl.ds(base_t, TPT), pl.ds(j * DC, DC)],
                rows_v, sem1)
            cp_z.wait()
            cp_l.wait()
            plsc.subcore_barrier()
            c1 = pltpu.async_copy(rows_v, shared.at[idx1_v], sem0, add=True)
            c2 = pltpu.async_copy(rows_v, shared.at[idx2_v], sem1, add=True)
            c1.wait()
            c2.wait()
            plsc.subcore_barrier()
            pending = pltpu.async_copy(
                shared.at[pl.ds(wbase, ORW)],
                ei_hbm.at[pl.ds(wbase, ORW), pl.ds(j * DC, DC)], semr)
        pending.wait()

    return _dispatch


def _rnd16(v):
    u = lax.bitcast_convert_type(v, jnp.uint32)
    r = u + jnp.uint32(0x7FFF) + ((u >> 16) & jnp.uint32(1))
    return r >> 16


def _pack16(lo, hi):
    w = _rnd16(lo) | (_rnd16(hi) << 16)
    return lax.bitcast_convert_type(w, jnp.int32)


def _unpack16(p):
    w = lax.bitcast_convert_type(p, jnp.uint32)
    lo = lax.bitcast_convert_type(w << 16, jnp.float32)
    hi = lax.bitcast_convert_type(w & jnp.uint32(0xFFFF0000), jnp.float32)
    return lo, hi

def _ffn_body(ei_ref, w1_ref, w2_ref, w3_ref, out_ref, acc_ref):
    hb = pl.program_id(1)
    x = ei_ref[...]
    h1 = lax.dot_general(x, w1_ref[0], (((1,), (0,)), ((), ())),
                         preferred_element_type=jnp.float32)
    h2 = lax.dot_general(x, w2_ref[0], (((1,), (0,)), ((), ())),
                         preferred_element_type=jnp.float32)
    h = h1 * (1.0 / (1.0 + jnp.exp(-h1))) * h2
    part = lax.dot_general(h, w3_ref[0], (((1,), (0,)), ((), ())),
                           preferred_element_type=jnp.float32)

    @pl.when(hb == 0)
    def _():
        acc_ref[...] = part

    @pl.when(hb != 0)
    def _():
        tot = acc_ref[...] + part
        out_ref[...] = _pack16(tot[:, :D // 2], tot[:, D // 2:])


_ffn = pl.pallas_call(
    _ffn_body,
    grid=(NE, H // BH),
    in_specs=[
        pl.BlockSpec((CAP, D), lambda e, hb: (e, 0)),
        pl.BlockSpec((1, D, BH), lambda e, hb: (e, 0, hb)),
        pl.BlockSpec((1, D, BH), lambda e, hb: (e, 0, hb)),
        pl.BlockSpec((1, BH, D), lambda e, hb: (e, hb, 0)),
    ],
    out_specs=pl.BlockSpec((CAP, D // 2), lambda e, hb: (e, 0)),
    out_shape=jax.ShapeDtypeStruct((SLOTS, D // 2), jnp.int32),
    scratch_shapes=[pltpu.VMEM((CAP, D), jnp.float32)],
)


@functools.cache
def _make_combine_gather():
    mesh = plsc.VectorSubcoreMesh(core_axis_name="c", subcore_axis_name="s")

    @functools.partial(
        pl.kernel,
        mesh=mesh,
        out_type=jax.ShapeDtypeStruct((2 * T, D // 2), jnp.int32),
        scratch_types=[
            pltpu.VMEM((TPW,), jnp.int32),
            pltpu.VMEM((TPW, D // 2), jnp.int32),
            pltpu.SemaphoreType.DMA,
        ],
    )
    def _combine_gather(eo_hbm, ga1_hbm, ga2_hbm, g_hbm, idx_v, rows_v, sem):
        c = lax.axis_index("c")
        s = lax.axis_index("s")
        wid = s * 2 + c
        base = wid * TPW
        pltpu.sync_copy(ga1_hbm.at[pl.ds(base, TPW)], idx_v)
        pltpu.async_copy(eo_hbm.at[idx_v], rows_v, sem).wait()
        pltpu.sync_copy(rows_v, g_hbm.at[pl.ds(base, TPW)])
        pltpu.sync_copy(ga2_hbm.at[pl.ds(base, TPW)], idx_v)
        pltpu.async_copy(eo_hbm.at[idx_v], rows_v, sem).wait()
        pltpu.sync_copy(rows_v, g_hbm.at[pl.ds(T + base, TPW)])

    return _combine_gather


def _wadd_body(g_ref, cw1_ref, cw2_ref, out_ref):
    lo1, hi1 = _unpack16(g_ref[0])
    lo2, hi2 = _unpack16(g_ref[1])
    out_ref[:, :D // 2] = cw1_ref[...] * lo1 + cw2_ref[...] * lo2
    out_ref[:, D // 2:] = cw1_ref[...] * hi1 + cw2_ref[...] * hi2


_wadd = pl.pallas_call(
    _wadd_body,
    out_shape=jax.ShapeDtypeStruct((T, D), jnp.float32),
)


def kernel(x, norm_g, gate_w, w1, w2, w3):
    b, s, d = x.shape
    xf = x.reshape(T, D)
    xn, da1, da2, ga1, ga2, cw1, cw2 = _route(xf, norm_g.reshape(1, D), gate_w)
    zeros = jnp.zeros((SLOTS, DC), jnp.float32)
    ei = _make_dispatch()(xn, da1.reshape(T), da2.reshape(T), zeros)
    eo = _ffn(ei, w1, w2, w3)
    g = _make_combine_gather()(eo, ga1.reshape(T), ga2.reshape(T))
    out = _wadd(g.reshape(2, T, D // 2), cw1.reshape(T, 1), cw2.reshape(T, 1))
    return out.reshape(b, s, d)
